# baseline (speedup 1.0000x reference)
import functools

import jax
import jax.numpy as jnp
from jax import lax
from jax.experimental import pallas as pl
from jax.experimental.pallas import tpu as pltpu
from jax.experimental.pallas import tpu_sc as plsc

N_OBS = 1000000
NC = 2
NS = 16
NW = NC * NS
LANES = 16

CHUNK = 3968
NCHUNKS = (N_OBS + CHUNK - 1) // CHUNK
EXT_WORDS = 2000 * 7


def _sc_body(p2d, ptidx, imidx, dep, par, pts, ext,
             out,
             ptidx_v, imidx_v, pxi_v, pyi_v, pzi_v, pxv, pyv, pzv, p2dv,
             depv, out_v, ext_v, par_v, semi, semg):
    wid = lax.axis_index("s") * NC + lax.axis_index("c")
    pltpu.sync_copy(par, par_v)
    pltpu.sync_copy(ext, ext_v)
    fx = par_v[0]
    fy = par_v[1]
    k1 = par_v[2]
    k2 = par_v[3]
    ppx = par_v[4]
    ppy = par_v[5]

    iota = lax.iota(jnp.int32, LANES)
    c0 = iota * 0
    c1 = c0 + 1
    c2 = c0 + 2

    nj = (NCHUNKS - 1 - wid) // NW + 1

    def chunk_body(j, carry):
        c = wid + j * NW
        base = jnp.minimum(c * CHUNK, N_OBS - CHUNK)
        cpa = pltpu.async_copy(ptidx.at[pl.ds(base, CHUNK)], ptidx_v, semi)
        cpb = pltpu.async_copy(imidx.at[pl.ds(base, CHUNK)], imidx_v, semi)
        cpc = pltpu.async_copy(p2d.at[pl.ds(base * 2, CHUNK * 2)], p2dv, semi)
        cpd = pltpu.async_copy(dep.at[pl.ds(base, CHUNK)], depv, semi)
        cpa.wait()

        def prep(g, pc):
            sl = pl.ds(g * LANES, LANES)
            t = ptidx_v[sl] * 3
            pxi_v[sl] = t
            pyi_v[sl] = t + 1
            pzi_v[sl] = t + 2
            return pc

        lax.fori_loop(0, CHUNK // LANES, prep, 0)
        cp1 = pltpu.async_copy(pts.at[pxi_v], pxv, semg)
        cp2 = pltpu.async_copy(pts.at[pyi_v], pyv, semg)
        cp3 = pltpu.async_copy(pts.at[pzi_v], pzv, semg)
        cpb.wait()
        cpc.wait()
        cpd.wait()
        cp1.wait()
        cp2.wait()
        cp3.wait()

        def grp(g, gc):
            b = g * LANES
            sl = pl.ds(b, LANES)
            rI = b + iota
            vx = pxv[sl]
            vy = pyv[sl]
            vz = pzv[sl]
            im7 = imidx_v[sl] * 7
            qw = plsc.load_gather(ext_v, [im7])
            qx = plsc.load_gather(ext_v, [im7 + 1])
            qy = plsc.load_gather(ext_v, [im7 + 2])
            qz = plsc.load_gather(ext_v, [im7 + 3])
            tx = plsc.load_gather(ext_v, [im7 + 4])
            ty = plsc.load_gather(ext_v, [im7 + 5])
            tz = plsc.load_gather(ext_v, [im7 + 6])
            rI2 = rI * 2
            ox = plsc.load_gather(p2dv, [rI2])
            oy = plsc.load_gather(p2dv, [rI2 + 1])
            dref = depv[sl]

            s = qw * qw + qx * qx + qy * qy + qz * qz
            uvx = qy * vz - qz * vy
            uvy = qz * vx - qx * vz
            uvz = qx * vy - qy * vx
            uux = qy * uvz - qz * uvy
            uuy = qz * uvx - qx * uvz
            uuz = qx * uvy - qy * uvx
            inv2 = 2.0 / s
            pcx = vx + inv2 * (qw * uvx + uux) + tx
            pcy = vy + inv2 * (qw * uvy + uuy) + ty
            pcz = vz + inv2 * (qw * uvz + uuz) + tz
            rcp = 1.0 / (pcz + 1e-6)
            xn = pcx * rcp
            yn = pcy * rcp
            r2 = xn * xn + yn * yn
            dist = 1.0 + r2 * (k1 + k2 * r2)
            rI3 = rI * 3
            plsc.store_scatter(out_v, [rI3], fx * xn * dist + ppx - ox)
            plsc.store_scatter(out_v, [rI3 + 1], fy * yn * dist + ppy - oy)
            plsc.store_scatter(out_v, [rI3 + 2], rcp - dref)
            return gc

        lax.fori_loop(0, CHUNK // LANES, grp, 0)
        pltpu.sync_copy(out_v, out.at[pl.ds(base * 3, CHUNK * 3)])
        return carry

    lax.fori_loop(0, nj, chunk_body, 0)


_sc_call = functools.partial(
    pl.kernel,
    out_type=jax.ShapeDtypeStruct((N_OBS * 3,), jnp.float32),
    mesh=plsc.VectorSubcoreMesh(core_axis_name="c", subcore_axis_name="s"),
    compiler_params=pltpu.CompilerParams(needs_layout_passes=False),
    scratch_types=[
        pltpu.VMEM((CHUNK,), jnp.int32),
        pltpu.VMEM((CHUNK,), jnp.int32),
        pltpu.VMEM((CHUNK,), jnp.int32),
        pltpu.VMEM((CHUNK,), jnp.int32),
        pltpu.VMEM((CHUNK,), jnp.int32),
        pltpu.VMEM((CHUNK,), jnp.float32),
        pltpu.VMEM((CHUNK,), jnp.float32),
        pltpu.VMEM((CHUNK,), jnp.float32),
        pltpu.VMEM((CHUNK * 2,), jnp.float32),
        pltpu.VMEM((CHUNK,), jnp.float32),
        pltpu.VMEM((CHUNK * 3,), jnp.float32),
        pltpu.VMEM((EXT_WORDS,), jnp.float32),
        pltpu.VMEM((6, LANES), jnp.float32),
        pltpu.SemaphoreType.DMA,
        pltpu.SemaphoreType.DMA,
    ],
)(_sc_body)


def kernel(points_2d, image_indices, camera_indices, point_indices,
           camera_pps, depths_ref, extrinsics, intrinsics, points_3d):
    ptidx = point_indices.astype(jnp.int32)
    imidx = image_indices.astype(jnp.int32)
    par = jnp.tile(
        jnp.concatenate([intrinsics[0], camera_pps[0]])[:, None], (1, LANES))
    out = _sc_call(points_2d.reshape(-1), ptidx, imidx, depths_ref, par,
                   points_3d.reshape(-1), extrinsics.reshape(-1))
    return out.reshape(N_OBS, 3)


# speedup vs baseline: 6.2590x; 6.2590x over previous
import functools

import jax
import jax.numpy as jnp
from jax import lax
from jax.experimental import pallas as pl
from jax.experimental.pallas import tpu as pltpu
from jax.experimental.pallas import tpu_sc as plsc

N_OBS = 1000000
NC = 2
NS = 16
NW = NC * NS
LANES = 16

CHUNK = 3968
NCHUNKS = (N_OBS + CHUNK - 1) // CHUNK
EXT_WORDS = 2000 * 7


def _sc_body(p2dx, p2dy, ptidx, imidx, dep, par, px, py, pz, ext,
             uo_out, vo_out, do_out,
             ptidx_v, imidx_v, pxv, pyv, pzv, oxv, oyv,
             depv, uov, vov, dov, ext_v, par_v, semi, semg):
    wid = lax.axis_index("s") * NC + lax.axis_index("c")
    pltpu.sync_copy(par, par_v)
    pltpu.sync_copy(ext, ext_v)
    fx = par_v[0]
    fy = par_v[1]
    k1 = par_v[2]
    k2 = par_v[3]
    ppx = par_v[4]
    ppy = par_v[5]

    iota = lax.iota(jnp.int32, LANES)
    c0 = iota * 0
    c1 = c0 + 1
    c2 = c0 + 2

    nj = (NCHUNKS - 1 - wid) // NW + 1

    def chunk_body(j, carry):
        c = wid + j * NW
        base = jnp.minimum(c * CHUNK, N_OBS - CHUNK)
        cpa = pltpu.async_copy(ptidx.at[pl.ds(base, CHUNK)], ptidx_v, semi)
        cpb = pltpu.async_copy(imidx.at[pl.ds(base, CHUNK)], imidx_v, semi)
        cpc = pltpu.async_copy(p2dx.at[pl.ds(base, CHUNK)], oxv, semi)
        cpd = pltpu.async_copy(p2dy.at[pl.ds(base, CHUNK)], oyv, semi)
        cpe = pltpu.async_copy(dep.at[pl.ds(base, CHUNK)], depv, semi)
        cpa.wait()
        cp1 = pltpu.async_copy(px.at[ptidx_v], pxv, semg)
        cp2 = pltpu.async_copy(py.at[ptidx_v], pyv, semg)
        cp3 = pltpu.async_copy(pz.at[ptidx_v], pzv, semg)
        cpb.wait()
        cpc.wait()
        cpd.wait()
        cpe.wait()
        cp1.wait()
        cp2.wait()
        cp3.wait()

        def grp(g, gc):
            b = g * LANES
            sl = pl.ds(b, LANES)
            rI = b + iota
            vx = pxv[sl]
            vy = pyv[sl]
            vz = pzv[sl]
            im7 = imidx_v[sl] * 7
            qw = plsc.load_gather(ext_v, [im7])
            qx = plsc.load_gather(ext_v, [im7 + 1])
            qy = plsc.load_gather(ext_v, [im7 + 2])
            qz = plsc.load_gather(ext_v, [im7 + 3])
            tx = plsc.load_gather(ext_v, [im7 + 4])
            ty = plsc.load_gather(ext_v, [im7 + 5])
            tz = plsc.load_gather(ext_v, [im7 + 6])
            ox = oxv[sl]
            oy = oyv[sl]
            dref = depv[sl]

            s = qw * qw + qx * qx + qy * qy + qz * qz
            uvx = qy * vz - qz * vy
            uvy = qz * vx - qx * vz
            uvz = qx * vy - qy * vx
            uux = qy * uvz - qz * uvy
            uuy = qz * uvx - qx * uvz
            uuz = qx * uvy - qy * uvx
            inv2 = 2.0 / s
            pcx = vx + inv2 * (qw * uvx + uux) + tx
            pcy = vy + inv2 * (qw * uvy + uuy) + ty
            pcz = vz + inv2 * (qw * uvz + uuz) + tz
            rcp = 1.0 / (pcz + 1e-6)
            xn = pcx * rcp
            yn = pcy * rcp
            r2 = xn * xn + yn * yn
            dist = 1.0 + r2 * (k1 + k2 * r2)
            uov[sl] = fx * xn * dist + ppx - ox
            vov[sl] = fy * yn * dist + ppy - oy
            dov[sl] = rcp - dref
            return gc

        lax.fori_loop(0, CHUNK // LANES, grp, 0)
        pltpu.sync_copy(uov, uo_out.at[pl.ds(base, CHUNK)])
        pltpu.sync_copy(vov, vo_out.at[pl.ds(base, CHUNK)])
        pltpu.sync_copy(dov, do_out.at[pl.ds(base, CHUNK)])
        return carry

    lax.fori_loop(0, nj, chunk_body, 0)


_sc_call = functools.partial(
    pl.kernel,
    out_type=(
        jax.ShapeDtypeStruct((N_OBS,), jnp.float32),
        jax.ShapeDtypeStruct((N_OBS,), jnp.float32),
        jax.ShapeDtypeStruct((N_OBS,), jnp.float32),
    ),
    mesh=plsc.VectorSubcoreMesh(core_axis_name="c", subcore_axis_name="s"),
    compiler_params=pltpu.CompilerParams(needs_layout_passes=False),
    scratch_types=[
        pltpu.VMEM((CHUNK,), jnp.int32),
        pltpu.VMEM((CHUNK,), jnp.int32),
        pltpu.VMEM((CHUNK,), jnp.float32),
        pltpu.VMEM((CHUNK,), jnp.float32),
        pltpu.VMEM((CHUNK,), jnp.float32),
        pltpu.VMEM((CHUNK,), jnp.float32),
        pltpu.VMEM((CHUNK,), jnp.float32),
        pltpu.VMEM((CHUNK,), jnp.float32),
        pltpu.VMEM((CHUNK,), jnp.float32),
        pltpu.VMEM((CHUNK,), jnp.float32),
        pltpu.VMEM((CHUNK,), jnp.float32),
        pltpu.VMEM((EXT_WORDS,), jnp.float32),
        pltpu.VMEM((6, LANES), jnp.float32),
        pltpu.SemaphoreType.DMA,
        pltpu.SemaphoreType.DMA,
    ],
)(_sc_body)


def kernel(points_2d, image_indices, camera_indices, point_indices,
           camera_pps, depths_ref, extrinsics, intrinsics, points_3d):
    ptidx = point_indices.astype(jnp.int32)
    imidx = image_indices.astype(jnp.int32)
    par = jnp.tile(
        jnp.concatenate([intrinsics[0], camera_pps[0]])[:, None], (1, LANES))
    uo, vo, do = _sc_call(points_2d[:, 0], points_2d[:, 1], ptidx, imidx,
                          depths_ref, par, points_3d[:, 0], points_3d[:, 1],
                          points_3d[:, 2], extrinsics.reshape(-1))
    return jnp.stack([uo, vo, do], axis=-1)


# speedup vs baseline: 7.6194x; 1.2174x over previous
import functools

import jax
import jax.numpy as jnp
from jax import lax
from jax.experimental import pallas as pl
from jax.experimental.pallas import tpu as pltpu
from jax.experimental.pallas import tpu_sc as plsc

N_OBS = 1000000
NC = 2
NS = 16
NW = NC * NS
LANES = 16

CHUNK = 3968
NJ = 8
EXT_WORDS = 2000 * 7
NGRP = CHUNK // LANES


def _sc_body(p2dx, p2dy, ptidx, imidx, dep, par, px, py, pz, ext,
             uo_out, vo_out, do_out,
             pt0, pt1, im0, im1, px0, px1, py0, py1, pz0, pz1,
             ox0, ox1, oy0, oy1, dp0, dp1, uo0, uo1, vo0, vo1, do0, do1,
             ext_v, par_v, si0, si1, sg0, sg1, so0, so1):
    wid = lax.axis_index("s") * NC + lax.axis_index("c")
    pltpu.sync_copy(par, par_v)
    pltpu.sync_copy(ext, ext_v)
    fx = par_v[0]
    fy = par_v[1]
    k1 = par_v[2]
    k2 = par_v[3]
    ppx = par_v[4]
    ppy = par_v[5]

    bufs = (
        (pt0, im0, px0, py0, pz0, ox0, oy0, dp0, uo0, vo0, do0, si0, sg0, so0),
        (pt1, im1, px1, py1, pz1, ox1, oy1, dp1, uo1, vo1, do1, si1, sg1, so1),
    )

    def base_of(j):
        c = wid + j * NW
        return jnp.minimum(c * CHUNK, N_OBS - CHUNK)

    def issue_inputs(j, b):
        base = base_of(j)
        pt, im, _, _, _, ox, oy, dp, _, _, _, si, _, _ = b
        sl = pl.ds(base, CHUNK)
        return (pltpu.async_copy(ptidx.at[sl], pt, si),
                pltpu.async_copy(imidx.at[sl], im, si),
                pltpu.async_copy(p2dx.at[sl], ox, si),
                pltpu.async_copy(p2dy.at[sl], oy, si),
                pltpu.async_copy(dep.at[sl], dp, si))

    def issue_gathers(b):
        pt, _, pxv, pyv, pzv, _, _, _, _, _, _, _, sg, _ = b
        return (pltpu.async_copy(px.at[pt], pxv, sg),
                pltpu.async_copy(py.at[pt], pyv, sg),
                pltpu.async_copy(pz.at[pt], pzv, sg))

    def issue_outputs(j, b):
        base = base_of(j)
        _, _, _, _, _, _, _, _, uov, vov, dov, _, _, so = b
        sl = pl.ds(base, CHUNK)
        return (pltpu.async_copy(uov, uo_out.at[sl], so),
                pltpu.async_copy(vov, vo_out.at[sl], so),
                pltpu.async_copy(dov, do_out.at[sl], so))

    def compute(b):
        _, im, pxv, pyv, pzv, oxv, oyv, dpv, uov, vov, dov, _, _, _ = b

        def grp(g, gc):
            sl = pl.ds(g * LANES, LANES)
            vx = pxv[sl]
            vy = pyv[sl]
            vz = pzv[sl]
            im7 = im[sl] * 7
            qw = plsc.load_gather(ext_v, [im7])
            qx = plsc.load_gather(ext_v, [im7 + 1])
            qy = plsc.load_gather(ext_v, [im7 + 2])
            qz = plsc.load_gather(ext_v, [im7 + 3])
            tx = plsc.load_gather(ext_v, [im7 + 4])
            ty = plsc.load_gather(ext_v, [im7 + 5])
            tz = plsc.load_gather(ext_v, [im7 + 6])
            ox = oxv[sl]
            oy = oyv[sl]
            dref = dpv[sl]

            s = qw * qw + qx * qx + qy * qy + qz * qz
            uvx = qy * vz - qz * vy
            uvy = qz * vx - qx * vz
            uvz = qx * vy - qy * vx
            uux = qy * uvz - qz * uvy
            uuy = qz * uvx - qx * uvz
            uuz = qx * uvy - qy * uvx
            inv2 = 2.0 / s
            pcx = vx + inv2 * (qw * uvx + uux) + tx
            pcy = vy + inv2 * (qw * uvy + uuy) + ty
            pcz = vz + inv2 * (qw * uvz + uuz) + tz
            rcp = 1.0 / (pcz + 1e-6)
            xn = pcx * rcp
            yn = pcy * rcp
            r2 = xn * xn + yn * yn
            dist = 1.0 + r2 * (k1 + k2 * r2)
            uov[sl] = fx * xn * dist + ppx - ox
            vov[sl] = fy * yn * dist + ppy - oy
            dov[sl] = rcp - dref
            return gc

        lax.fori_loop(0, NGRP, grp, 0)

    pend_in = {0: issue_inputs(0, bufs[0]), 1: issue_inputs(1, bufs[1])}
    pend_g = {}
    pend_out = {}
    for cp in pend_in.pop(0):
        cp.wait()
    pend_g[0] = issue_gathers(bufs[0])
    for j in range(NJ):
        cur = bufs[j % 2]
        if j + 1 < NJ:
            for cp in pend_in.pop(j + 1):
                cp.wait()
            pend_g[j + 1] = issue_gathers(bufs[(j + 1) % 2])
        for cp in pend_g.pop(j):
            cp.wait()
        if j >= 2:
            for cp in pend_out.pop(j - 2):
                cp.wait()
        compute(cur)
        pend_out[j] = issue_outputs(j, cur)
        if j + 2 < NJ:
            pend_in[j + 2] = issue_inputs(j + 2, cur)
    for j in (NJ - 2, NJ - 1):
        for cp in pend_out.pop(j):
            cp.wait()


_sc_call = functools.partial(
    pl.kernel,
    out_type=(
        jax.ShapeDtypeStruct((N_OBS,), jnp.float32),
        jax.ShapeDtypeStruct((N_OBS,), jnp.float32),
        jax.ShapeDtypeStruct((N_OBS,), jnp.float32),
    ),
    mesh=plsc.VectorSubcoreMesh(core_axis_name="c", subcore_axis_name="s"),
    compiler_params=pltpu.CompilerParams(needs_layout_passes=False),
    scratch_types=(
        [pltpu.VMEM((CHUNK,), jnp.int32) for _ in range(4)]
        + [pltpu.VMEM((CHUNK,), jnp.float32) for _ in range(18)]
        + [pltpu.VMEM((EXT_WORDS,), jnp.float32),
           pltpu.VMEM((6, LANES), jnp.float32)]
        + [pltpu.SemaphoreType.DMA for _ in range(6)]
    ),
)(_sc_body)


def kernel(points_2d, image_indices, camera_indices, point_indices,
           camera_pps, depths_ref, extrinsics, intrinsics, points_3d):
    ptidx = point_indices.astype(jnp.int32)
    imidx = image_indices.astype(jnp.int32)
    par = jnp.tile(
        jnp.concatenate([intrinsics[0], camera_pps[0]])[:, None], (1, LANES))
    uo, vo, do = _sc_call(points_2d[:, 0], points_2d[:, 1], ptidx, imidx,
                          depths_ref, par, points_3d[:, 0], points_3d[:, 1],
                          points_3d[:, 2], extrinsics.reshape(-1))
    return jnp.stack([uo, vo, do], axis=-1)
